# Initial kernel scaffold; baseline (speedup 1.0000x reference)
import jax
import jax.numpy as jnp
from jax.experimental import pallas as pl


def kernel(u, v, usr_feat, item_feat, rel_feat, neigh_uu, neigh_uu_st, neigh_ui, neigh_ui_rat, neigh_ui_vot, neigh_ui_tim, neigh_iu, neigh_iu_rat, neigh_iu_vot, neigh_iu_tim, neigh_ii, neigh_ir, W_u, b_u, W_v, b_v):
    raise NotImplementedError("write your pallas kernel here")


# speedup vs baseline: 1.0869x; 1.0869x over previous
import jax
import jax.numpy as jnp
from jax import lax
from jax.experimental import pallas as pl
from jax.experimental.pallas import tpu as pltpu
from jax.experimental.pallas import tpu_sc as plsc

NC = 2
NS = 16
L = 16
NW = NC * NS

B = 16384
D = 16
S = 8
E = B // NW
C = 16
NCH = E // C
R = 512


def _sc_body(u_hbm, v_hbm, uh_hbm, up_hbm, vh_hbm, vp_hbm,
             usr_hbm, item_hbm, rel_hbm,
             nuu_hbm, nui_hbm, niu_hbm, nii_hbm, nir_hbm,
             st_hbm, ratui_hbm, votui_hbm, timui_hbm,
             ratiu_hbm, votiu_hbm, timiu_hbm,
             uu_out, ui_out, iu_out, ii_out, ir_out,
             uself_out, vself_out,
             st_out, ratui_out, votui_out, timui_out,
             ratiu_out, votiu_out, timiu_out,
             uc, vc, uhc, upc, vhc, vpc,
             iuu, iui, iiu, iii, iir,
             est, eratui, evotui, etimui, eratiu, evotiu, etimiu,
             uself_v, vself_v,
             fx_uu, fx_ui, fx_iu, fx_ii, fx_ir,
             ruu, rui, riu, rii, rir,
             sem1, sem2, semo):
    wid = lax.axis_index("s") * NC + lax.axis_index("c")
    base = pl.multiple_of(wid * E, E)
    iota = lax.iota(jnp.int32, L)

    def chunk_body(c, _):
        off = pl.multiple_of(base + c * C, C)
        pltpu.sync_copy(u_hbm.at[pl.ds(off, C)], uc)
        pltpu.sync_copy(v_hbm.at[pl.ds(off, C)], vc)
        pltpu.sync_copy(uh_hbm.at[pl.ds(off, C)], uhc)
        pltpu.sync_copy(up_hbm.at[pl.ds(off, C)], upc)
        pltpu.sync_copy(vh_hbm.at[pl.ds(off, C)], vhc)
        pltpu.sync_copy(vp_hbm.at[pl.ds(off, C)], vpc)

        l1 = [
            (usr_hbm, uc, uself_v), (item_hbm, vc, vself_v),
            (nuu_hbm, uhc, iuu), (nui_hbm, uhc, iui),
            (niu_hbm, vhc, iiu), (nii_hbm, vhc, iii), (nir_hbm, vhc, iir),
            (st_hbm, uhc, est),
            (ratui_hbm, uhc, eratui), (votui_hbm, uhc, evotui),
            (timui_hbm, uhc, etimui),
            (ratiu_hbm, vhc, eratiu), (votiu_hbm, vhc, evotiu),
            (timiu_hbm, vhc, etimiu),
        ]
        for tab, idx, dst in l1:
            pltpu.async_copy(tab.at[idx], dst, sem1)
        for tab, idx, dst in l1:
            pltpu.make_async_copy(tab.at[idx], dst, sem1).wait()

        outs = [
            (uself_v, uself_out), (vself_v, vself_out),
            (est, st_out), (eratui, ratui_out), (evotui, votui_out),
            (etimui, timui_out), (eratiu, ratiu_out), (evotiu, votiu_out),
            (etimiu, timiu_out),
        ]
        for src, dst in outs:
            pltpu.async_copy(src, dst.at[pl.ds(off, C)], semo)

        upv = upc[pl.ds(0, L)]
        vpv = vpc[pl.ds(0, L)]
        for parv, pairs in ((upv, ((iuu, fx_uu), (iui, fx_ui))),
                            (vpv, ((iiu, fx_iu), (iii, fx_ii),
                                   (iir, fx_ir)))):
            for j in range(C // 2):
                e0, e1 = 2 * j, 2 * j + 1
                p0 = parv.at[jnp.full((L,), e0, jnp.int32)].get(
                    mode="promise_in_bounds")
                p1 = parv.at[jnp.full((L,), e1, jnp.int32)].get(
                    mode="promise_in_bounds")
                ia = (iota + 8 * p0) & 15
                ib = (iota - 8 + 8 * p1) & 15
                for src, fx in pairs:
                    a = src[e0, :].at[ia].get(mode="promise_in_bounds")
                    b = src[e1, :].at[ib].get(mode="promise_in_bounds")
                    fx[pl.ds(j * L, L)] = jnp.where(iota < 8, a, b)

        l2 = [
            (usr_hbm, fx_uu, ruu, uu_out), (item_hbm, fx_ui, rui, ui_out),
            (usr_hbm, fx_iu, riu, iu_out), (item_hbm, fx_ii, rii, ii_out),
            (rel_hbm, fx_ir, rir, ir_out),
        ]
        for tab, fx, dst, _o in l2:
            pltpu.async_copy(tab.at[fx], dst, sem2)
        for tab, fx, dst, _o in l2:
            pltpu.make_async_copy(tab.at[fx], dst, sem2).wait()

        row8 = pl.multiple_of(off * S, C * S)
        for _t, _f, dst, out in l2:
            pltpu.async_copy(dst, out.at[pl.ds(row8, C * S)], semo)

        for src, dst in outs:
            pltpu.make_async_copy(src, dst.at[pl.ds(off, C)], semo).wait()
        for _t, _f, dst, out in l2:
            pltpu.make_async_copy(dst, out.at[pl.ds(row8, C * S)],
                                  semo).wait()
        return 0

    lax.fori_loop(0, NCH, chunk_body, 0, unroll=False)


def _sc_gather(u, v, uh, up, vh, vp, usr_feat, item_feat, rel_feat,
               nuu2, nui2, niu2, nii2, nir2,
               st2, ratui2, votui2, timui2, ratiu2, votiu2, timiu2):
    f32, i32 = jnp.float32, jnp.int32
    row = lambda n: jax.ShapeDtypeStruct((n, D), f32)
    kfn = pl.kernel(
        _sc_body,
        out_type=[row(B * S), row(B * S), row(B * S), row(B * S), row(B * S),
                  row(B), row(B),
                  row(B), row(B), row(B), row(B), row(B), row(B), row(B)],
        mesh=plsc.VectorSubcoreMesh(core_axis_name="c", subcore_axis_name="s"),
        compiler_params=pltpu.CompilerParams(use_tc_tiling_on_sc=False),
        scratch_types=(
            [pltpu.VMEM((C,), i32) for _ in range(6)]
            + [pltpu.VMEM((C, D), i32) for _ in range(5)]
            + [pltpu.VMEM((C, D), f32) for _ in range(7)]
            + [pltpu.VMEM((C, D), f32) for _ in range(2)]
            + [pltpu.VMEM((C * S,), i32) for _ in range(5)]
            + [pltpu.VMEM((C * S, D), f32) for _ in range(5)]
            + [pltpu.SemaphoreType.DMA for _ in range(3)]
        ),
    )
    return kfn(u, v, uh, up, vh, vp, usr_feat, item_feat, rel_feat,
               nuu2, nui2, niu2, nii2, nir2,
               st2, ratui2, votui2, timui2, ratiu2, votiu2, timiu2)


def _tc_body(pu_ref, pv_ref,
             uu_ref, ui_ref, iu_ref, ii_ref, ir_ref,
             uself_ref, vself_ref,
             st_ref, ratui_ref, votui_ref, timui_ref,
             ratiu_ref, votiu_ref, timiu_ref,
             wu_ref, bu_ref, wv_ref, bv_ref,
             out_ref):
    pu = pu_ref[...]
    pv = pv_ref[...]

    def sel(x, par):
        return jnp.where(par == 1, x[:, 8:16], x[:, 0:8])

    uself = uself_ref[...]
    vself = vself_ref[...]

    def wsum(w, rows_ref):
        rows = rows_ref[...]
        acc = w[:, 0:1] * rows[:, 0:D]
        for s in range(1, S):
            acc = acc + w[:, s:s + 1] * rows[:, s * D:(s + 1) * D]
        return acc

    def softmax8(z):
        m = jnp.max(z, axis=1, keepdims=True)
        e = jnp.exp(z - m)
        return e / jnp.sum(e, axis=1, keepdims=True)

    st = jax.nn.sigmoid(sel(st_ref[...], pu))
    uu_agg = wsum(st, uu_ref) * (1.0 / S)
    z_ui = sel(ratui_ref[...], pu) * sel(votui_ref[...], pu) \
        + sel(timui_ref[...], pu)
    ui_agg = wsum(softmax8(z_ui), ui_ref)
    u_vec = jnp.tanh(
        jnp.dot(uself + uu_agg + ui_agg, wu_ref[...],
                preferred_element_type=jnp.float32) + bu_ref[...])

    z_iu = sel(ratiu_ref[...], pv) * sel(votiu_ref[...], pv) \
        + sel(timiu_ref[...], pv)
    iu_agg = wsum(softmax8(z_iu), iu_ref)
    ir = ir_ref[...]
    pi = jnp.concatenate(
        [jnp.sum(ir[:, s * D:(s + 1) * D] * uself, axis=1, keepdims=True)
         for s in range(S)], axis=1)
    ii_agg = wsum(softmax8(pi), ii_ref)
    v_vec = jnp.tanh(
        jnp.dot(vself + iu_agg + ii_agg, wv_ref[...],
                preferred_element_type=jnp.float32) + bv_ref[...])

    sdot = jnp.sum(u_vec * v_vec, axis=1)
    out_ref[...] = 5.0 / (1.0 + jnp.exp(-sdot))


def _tc_math(pu, pv, uu, ui, iu, ii, ir, uself, vself,
             st, ratui, votui, timui, ratiu, votiu, timiu,
             W_u, b_u, W_v, b_v):
    n = B // R
    bs_r = lambda w: pl.BlockSpec((R, w), lambda i: (i, 0))
    bs_w = pl.BlockSpec((D, D), lambda i: (0, 0))
    bs_b = pl.BlockSpec((1, D), lambda i: (0, 0))
    return pl.pallas_call(
        _tc_body,
        grid=(n,),
        in_specs=[bs_r(1), bs_r(1),
                  bs_r(S * D), bs_r(S * D), bs_r(S * D), bs_r(S * D),
                  bs_r(S * D),
                  bs_r(D), bs_r(D),
                  bs_r(D), bs_r(D), bs_r(D), bs_r(D),
                  bs_r(D), bs_r(D), bs_r(D),
                  bs_w, bs_b, bs_w, bs_b],
        out_specs=pl.BlockSpec((R,), lambda i: (i,)),
        out_shape=jax.ShapeDtypeStruct((B,), jnp.float32),
    )(pu, pv, uu, ui, iu, ii, ir, uself, vself,
      st, ratui, votui, timui, ratiu, votiu, timiu,
      W_u, b_u, W_v, b_v)


def kernel(u, v, usr_feat, item_feat, rel_feat,
           neigh_uu, neigh_uu_st, neigh_ui, neigh_ui_rat, neigh_ui_vot,
           neigh_ui_tim, neigh_iu, neigh_iu_rat, neigh_iu_vot, neigh_iu_tim,
           neigh_ii, neigh_ir, W_u, b_u, W_v, b_v):
    i32 = jnp.int32
    u = u.astype(i32)
    v = v.astype(i32)
    half = lambda t: t.astype(i32).reshape(-1, 2 * S)
    halff = lambda t: t.reshape(-1, 2 * S)
    outs = _sc_gather(
        u, v, u >> 1, u & 1, v >> 1, v & 1,
        usr_feat, item_feat, rel_feat,
        half(neigh_uu), half(neigh_ui), half(neigh_iu),
        half(neigh_ii), half(neigh_ir),
        halff(neigh_uu_st), halff(neigh_ui_rat), halff(neigh_ui_vot),
        halff(neigh_ui_tim), halff(neigh_iu_rat), halff(neigh_iu_vot),
        halff(neigh_iu_tim))
    (uu, ui, iu, ii, ir, uself, vself,
     st, ratui, votui, timui, ratiu, votiu, timiu) = outs
    wide = lambda t: t.reshape(B, S * D)
    return _tc_math(
        (u & 1).reshape(B, 1), (v & 1).reshape(B, 1),
        wide(uu), wide(ui), wide(iu), wide(ii), wide(ir),
        uself, vself, st, ratui, votui, timui, ratiu, votiu, timiu,
        W_u, b_u.reshape(1, D), W_v, b_v.reshape(1, D))


# speedup vs baseline: 1.1093x; 1.0205x over previous
import jax
import jax.numpy as jnp
from jax import lax
from jax.experimental import pallas as pl
from jax.experimental.pallas import tpu as pltpu
from jax.experimental.pallas import tpu_sc as plsc

NC = 2
NS = 16
NW = NC * NS

B = 16384
D = 16
S = 8
E = B // NW
C = 16
NCH = E // C
R = 512


def _sc_body(u_hbm, v_hbm,
             usr_hbm, item_hbm, rel_hbm,
             nuu_hbm, nui_hbm, niu_hbm, nii_hbm, nir_hbm,
             st_hbm, ratui_hbm, votui_hbm, timui_hbm,
             ratiu_hbm, votiu_hbm, timiu_hbm,
             uu_out, ui_out, iu_out, ii_out, ir_out,
             uself_out, vself_out,
             st_out, ratui_out, votui_out, timui_out,
             ratiu_out, votiu_out, timiu_out,
             u_v, v_v,
             fx_uu, fx_ui, fx_iu, fx_ii, fx_ir,
             est, eratui, evotui, etimui, eratiu, evotiu, etimiu,
             uself_v, vself_v,
             ruu, rui, riu, rii, rir,
             sem1, sem2, semo):
    wid = lax.axis_index("s") * NC + lax.axis_index("c")
    base = pl.multiple_of(wid * E, E)

    pltpu.sync_copy(u_hbm.at[pl.ds(base, E)], u_v)
    pltpu.sync_copy(v_hbm.at[pl.ds(base, E)], v_v)

    def chunk_body(c, _):
        loc = pl.multiple_of(c * C, C)
        off = pl.multiple_of(base + c * C, C)
        uc = u_v.at[pl.ds(loc, C)]
        vc = v_v.at[pl.ds(loc, C)]

        l1 = [
            (usr_hbm, uc, uself_v), (item_hbm, vc, vself_v),
            (nuu_hbm, uc, fx_uu), (nui_hbm, uc, fx_ui),
            (niu_hbm, vc, fx_iu), (nii_hbm, vc, fx_ii),
            (nir_hbm, vc, fx_ir),
            (st_hbm, uc, est),
            (ratui_hbm, uc, eratui), (votui_hbm, uc, evotui),
            (timui_hbm, uc, etimui),
            (ratiu_hbm, vc, eratiu), (votiu_hbm, vc, evotiu),
            (timiu_hbm, vc, etimiu),
        ]
        for tab, idx, dst in l1:
            pltpu.async_copy(tab.at[idx], dst, sem1)
        for tab, idx, dst in l1:
            pltpu.make_async_copy(tab.at[idx], dst, sem1).wait()

        outs = [
            (uself_v, uself_out), (vself_v, vself_out),
            (est, st_out), (eratui, ratui_out), (evotui, votui_out),
            (etimui, timui_out), (eratiu, ratiu_out), (evotiu, votiu_out),
            (etimiu, timiu_out),
        ]
        for src, dst in outs:
            pltpu.async_copy(src, dst.at[pl.ds(off, C)], semo)

        l2 = [
            (usr_hbm, fx_uu, ruu, uu_out),
            (item_hbm, fx_ui, rui, ui_out),
            (usr_hbm, fx_iu, riu, iu_out),
            (item_hbm, fx_ii, rii, ii_out),
            (rel_hbm, fx_ir, rir, ir_out),
        ]
        for tab, fx, dst, _o in l2:
            for e in range(C):
                pltpu.async_copy(tab.at[fx.at[e]],
                                 dst.at[pl.ds(e * S, S)], sem2)
        for tab, fx, dst, _o in l2:
            for e in range(C):
                pltpu.make_async_copy(tab.at[fx.at[e]],
                                      dst.at[pl.ds(e * S, S)], sem2).wait()

        row8 = pl.multiple_of(off * S, C * S)
        for _t, _f, dst, out in l2:
            pltpu.async_copy(dst, out.at[pl.ds(row8, C * S)], semo)

        for src, dst in outs:
            pltpu.make_async_copy(src, dst.at[pl.ds(off, C)], semo).wait()
        for _t, _f, dst, out in l2:
            pltpu.make_async_copy(dst, out.at[pl.ds(row8, C * S)],
                                  semo).wait()
        return 0

    lax.fori_loop(0, NCH, chunk_body, 0, unroll=False)


def _sc_gather(u, v, usr_feat, item_feat, rel_feat,
               nuu, nui, niu, nii, nir,
               st, ratui, votui, timui, ratiu, votiu, timiu):
    f32, i32 = jnp.float32, jnp.int32
    row = lambda n, w: jax.ShapeDtypeStruct((n, w), f32)
    kfn = pl.kernel(
        _sc_body,
        out_type=[row(B * S, D), row(B * S, D), row(B * S, D),
                  row(B * S, D), row(B * S, D),
                  row(B, D), row(B, D),
                  row(B, S), row(B, S), row(B, S), row(B, S),
                  row(B, S), row(B, S), row(B, S)],
        mesh=plsc.VectorSubcoreMesh(core_axis_name="c", subcore_axis_name="s"),
        compiler_params=pltpu.CompilerParams(use_tc_tiling_on_sc=False),
        scratch_types=(
            [pltpu.VMEM((E,), i32) for _ in range(2)]
            + [pltpu.VMEM((C, S), i32) for _ in range(5)]
            + [pltpu.VMEM((C, S), f32) for _ in range(7)]
            + [pltpu.VMEM((C, D), f32) for _ in range(2)]
            + [pltpu.VMEM((C * S, D), f32) for _ in range(5)]
            + [pltpu.SemaphoreType.DMA for _ in range(3)]
        ),
    )
    return kfn(u, v, usr_feat, item_feat, rel_feat,
               nuu, nui, niu, nii, nir,
               st, ratui, votui, timui, ratiu, votiu, timiu)


def _tc_body(uu_ref, ui_ref, iu_ref, ii_ref, ir_ref,
             uself_ref, vself_ref,
             st_ref, ratui_ref, votui_ref, timui_ref,
             ratiu_ref, votiu_ref, timiu_ref,
             wu_ref, bu_ref, wv_ref, bv_ref,
             out_ref):
    uself = uself_ref[...]
    vself = vself_ref[...]

    def wsum(w, rows_ref):
        rows = rows_ref[...]
        acc = w[:, 0:1] * rows[:, 0:D]
        for s in range(1, S):
            acc = acc + w[:, s:s + 1] * rows[:, s * D:(s + 1) * D]
        return acc

    def softmax8(z):
        m = jnp.max(z, axis=1, keepdims=True)
        e = jnp.exp(z - m)
        return e / jnp.sum(e, axis=1, keepdims=True)

    st = jax.nn.sigmoid(st_ref[...])
    uu_agg = wsum(st, uu_ref) * (1.0 / S)
    z_ui = ratui_ref[...] * votui_ref[...] + timui_ref[...]
    ui_agg = wsum(softmax8(z_ui), ui_ref)
    u_vec = jnp.tanh(
        jnp.dot(uself + uu_agg + ui_agg, wu_ref[...],
                preferred_element_type=jnp.float32) + bu_ref[...])

    z_iu = ratiu_ref[...] * votiu_ref[...] + timiu_ref[...]
    iu_agg = wsum(softmax8(z_iu), iu_ref)
    ir = ir_ref[...]
    pi = jnp.concatenate(
        [jnp.sum(ir[:, s * D:(s + 1) * D] * uself, axis=1, keepdims=True)
         for s in range(S)], axis=1)
    ii_agg = wsum(softmax8(pi), ii_ref)
    v_vec = jnp.tanh(
        jnp.dot(vself + iu_agg + ii_agg, wv_ref[...],
                preferred_element_type=jnp.float32) + bv_ref[...])

    sdot = jnp.sum(u_vec * v_vec, axis=1)
    out_ref[...] = 5.0 / (1.0 + jnp.exp(-sdot))


def _tc_math(uu, ui, iu, ii, ir, uself, vself,
             st, ratui, votui, timui, ratiu, votiu, timiu,
             W_u, b_u, W_v, b_v):
    n = B // R
    bs_r = lambda w: pl.BlockSpec((R, w), lambda i: (i, 0))
    bs_w = pl.BlockSpec((D, D), lambda i: (0, 0))
    bs_b = pl.BlockSpec((1, D), lambda i: (0, 0))
    return pl.pallas_call(
        _tc_body,
        grid=(n,),
        in_specs=[bs_r(S * D), bs_r(S * D), bs_r(S * D), bs_r(S * D),
                  bs_r(S * D),
                  bs_r(D), bs_r(D),
                  bs_r(S), bs_r(S), bs_r(S), bs_r(S),
                  bs_r(S), bs_r(S), bs_r(S),
                  bs_w, bs_b, bs_w, bs_b],
        out_specs=pl.BlockSpec((R,), lambda i: (i,)),
        out_shape=jax.ShapeDtypeStruct((B,), jnp.float32),
    )(uu, ui, iu, ii, ir, uself, vself,
      st, ratui, votui, timui, ratiu, votiu, timiu,
      W_u, b_u, W_v, b_v)


def kernel(u, v, usr_feat, item_feat, rel_feat,
           neigh_uu, neigh_uu_st, neigh_ui, neigh_ui_rat, neigh_ui_vot,
           neigh_ui_tim, neigh_iu, neigh_iu_rat, neigh_iu_vot, neigh_iu_tim,
           neigh_ii, neigh_ir, W_u, b_u, W_v, b_v):
    i32 = jnp.int32
    outs = _sc_gather(
        u.astype(i32), v.astype(i32),
        usr_feat, item_feat, rel_feat,
        neigh_uu.astype(i32), neigh_ui.astype(i32), neigh_iu.astype(i32),
        neigh_ii.astype(i32), neigh_ir.astype(i32),
        neigh_uu_st, neigh_ui_rat, neigh_ui_vot, neigh_ui_tim,
        neigh_iu_rat, neigh_iu_vot, neigh_iu_tim)
    (uu, ui, iu, ii, ir, uself, vself,
     st, ratui, votui, timui, ratiu, votiu, timiu) = outs
    wide = lambda t: t.reshape(B, S * D)
    return _tc_math(
        wide(uu), wide(ui), wide(iu), wide(ii), wide(ir),
        uself, vself, st, ratui, votui, timui, ratiu, votiu, timiu,
        W_u, b_u.reshape(1, D), W_v, b_v.reshape(1, D))
